# Initial kernel scaffold; baseline (speedup 1.0000x reference)
import jax
import jax.numpy as jnp
from jax.experimental import pallas as pl


def kernel(x, edge_index, edge_weight, W0, b0, W1, b1):
    raise NotImplementedError("write your pallas kernel here")


# speedup vs baseline: 27.8285x; 27.8285x over previous
import functools

import jax
import jax.numpy as jnp
from jax import lax
from jax.experimental import pallas as pl
from jax.experimental.pallas import tpu as pltpu
from jax.experimental.pallas import tpu_sc as plsc

_N = 10000
_E = 320000
_D = 128
_H = 16
_NC = 2
_NS = 16
_NW = _NC * _NS
_EPW = _E // _NW
_CH = 80
_CPW = _EPW // _CH
_NP = 10240
_RPT = _NP // _NS

_mesh = plsc.VectorSubcoreMesh(core_axis_name="c", subcore_axis_name="s")


def _deg_body(row_hbm, w_hbm, out_hbm, rowv, wv, zbuf, acc):
    c = lax.axis_index("c")
    s = lax.axis_index("s")

    def zb(i, carry):
        zbuf[pl.ds(i * 16, 16)] = jnp.zeros((16,), jnp.float32)
        return carry

    lax.fori_loop(0, _RPT // 16, zb, 0)
    pltpu.sync_copy(zbuf, acc.at[pl.ds(s * _RPT, _RPT)])
    wid = c * _NS + s
    pltpu.sync_copy(row_hbm.at[wid], rowv)
    pltpu.sync_copy(w_hbm.at[wid], wv)
    plsc.subcore_barrier()

    def body(j, carry):
        pltpu.sync_copy(wv.at[j], acc.at[rowv.at[j]], add=True)
        return carry

    lax.fori_loop(0, _CPW, body, 0)
    plsc.subcore_barrier()
    pltpu.sync_copy(acc.at[pl.ds(s * _RPT, _RPT)],
                    out_hbm.at[c, pl.ds(s * _RPT, _RPT)])


_deg_call = pl.kernel(
    _deg_body,
    out_type=jax.ShapeDtypeStruct((_NC, _NP), jnp.float32),
    mesh=_mesh,
    scratch_types=[
        pltpu.VMEM((_CPW, _CH), jnp.int32),
        pltpu.VMEM((_CPW, _CH), jnp.float32),
        pltpu.VMEM((_RPT,), jnp.float32),
        pltpu.VMEM_SHARED((_NP,), jnp.float32),
    ],
)


def _agg_body(hs_hbm, row_hbm, col_hbm, w_hbm, out_hbm,
              rowv, colv, wv, gbuf, zbuf, acc):
    c = lax.axis_index("c")
    s = lax.axis_index("s")

    def zb(i, carry):
        zbuf[i, :] = jnp.zeros((_H,), jnp.float32)
        return carry

    lax.fori_loop(0, _RPT, zb, 0)
    pltpu.sync_copy(zbuf, acc.at[pl.ds(s * _RPT, _RPT)])
    wid = c * _NS + s
    pltpu.sync_copy(row_hbm.at[wid], rowv)
    pltpu.sync_copy(col_hbm.at[wid], colv)
    pltpu.sync_copy(w_hbm.at[wid], wv)
    plsc.subcore_barrier()

    def body(j, carry):
        pltpu.sync_copy(hs_hbm.at[rowv.at[j]], gbuf)
        for grp in range(_CH // 16):
            wvec = wv[j, pl.ds(grp * 16, 16)]
            for k in range(16):
                e = grp * 16 + k
                gbuf[e, :] = gbuf[e, :] * wvec[k]
        pltpu.sync_copy(gbuf, acc.at[colv.at[j]], add=True)
        return carry

    lax.fori_loop(0, _CPW, body, 0)
    plsc.subcore_barrier()
    pltpu.sync_copy(acc.at[pl.ds(s * _RPT, _RPT)],
                    out_hbm.at[c, pl.ds(s * _RPT, _RPT)])


_agg_call = pl.kernel(
    _agg_body,
    out_type=jax.ShapeDtypeStruct((_NC, _NP, _H), jnp.float32),
    mesh=_mesh,
    compiler_params=pltpu.CompilerParams(use_tc_tiling_on_sc=False),
    scratch_types=[
        pltpu.VMEM((_CPW, _CH), jnp.int32),
        pltpu.VMEM((_CPW, _CH), jnp.int32),
        pltpu.VMEM((_CPW, _CH), jnp.float32),
        pltpu.VMEM((_CH, _H), jnp.float32),
        pltpu.VMEM((_RPT, _H), jnp.float32),
        pltpu.VMEM_SHARED((_NP, _H), jnp.float32),
    ],
)


def _tc1_body(x_ref, w0_ref, dp_ref, hs_ref, g16_ref):
    deg = dp_ref[0, :_N] + dp_ref[1, :_N] + 1.0
    g = lax.rsqrt(deg)[:, None]
    h = jnp.dot(x_ref[...], w0_ref[...], preferred_element_type=jnp.float32)
    hs_ref[...] = h * g
    g16_ref[...] = jnp.broadcast_to(g, (_N, _H))


_tc1_call = pl.pallas_call(
    _tc1_body,
    out_shape=(jax.ShapeDtypeStruct((_N, _H), jnp.float32),
               jax.ShapeDtypeStruct((_N, _H), jnp.float32)),
)


def _tc2_body(ap_ref, hs_ref, g16_ref, w1_ref, b0_ref, hs2_ref):
    aggs = ap_ref[0, :_N, :] + ap_ref[1, :_N, :]
    out1 = jnp.maximum(
        g16_ref[...] * (aggs + hs_ref[...]) + b0_ref[...], 0.0)
    h2 = jnp.dot(out1, w1_ref[...], preferred_element_type=jnp.float32)
    hs2_ref[...] = h2 * g16_ref[...]


_tc2_call = pl.pallas_call(
    _tc2_body,
    out_shape=jax.ShapeDtypeStruct((_N, _H), jnp.float32),
)


def _tc3_body(qp_ref, hs2_ref, g16_ref, b1_ref, out_ref):
    aggs = qp_ref[0, :_N, :] + qp_ref[1, :_N, :]
    out_ref[...] = g16_ref[...] * (aggs + hs2_ref[...]) + b1_ref[...]


_tc3_call = pl.pallas_call(
    _tc3_body,
    out_shape=jax.ShapeDtypeStruct((_N, _H), jnp.float32),
)


def kernel(x, edge_index, edge_weight, W0, b0, W1, b1):
    row = edge_index[0].reshape(_NW, _CPW, _CH)
    col = edge_index[1].reshape(_NW, _CPW, _CH)
    w = edge_weight.reshape(_NW, _CPW, _CH)
    w1p = jnp.zeros((_H, _H), jnp.float32).at[:, : W1.shape[1]].set(W1)
    b1p = jnp.zeros((_H,), jnp.float32).at[: b1.shape[0]].set(b1)

    dpart = _deg_call(row, w)
    hs, g16 = _tc1_call(x, W0, dpart)
    ap = _agg_call(hs, row, col, w)
    hs2 = _tc2_call(ap, hs, g16, w1p, b0)
    qp = _agg_call(hs2, row, col, w)
    outp = _tc3_call(qp, hs2, g16, b1p)
    return outp[:, : W1.shape[1]]


# speedup vs baseline: 54.3040x; 1.9514x over previous
import functools

import jax
import jax.numpy as jnp
from jax import lax
from jax.experimental import pallas as pl
from jax.experimental.pallas import tpu as pltpu
from jax.experimental.pallas import tpu_sc as plsc

_N = 10000
_E = 320000
_D = 128
_H = 16
_NC = 2
_NS = 16
_NW = _NC * _NS
_EPW = _E // _NW
_CH = 80
_CPW = _EPW // _CH
_NP = 10240
_RPT = _NP // _NS

_mesh = plsc.VectorSubcoreMesh(core_axis_name="c", subcore_axis_name="s")


def _deg_body(row_hbm, w_hbm, out_hbm, rowv, wv, zbuf, acc):
    c = lax.axis_index("c")
    s = lax.axis_index("s")

    def zb(i, carry):
        zbuf[pl.ds(i * 16, 16)] = jnp.zeros((16,), jnp.float32)
        return carry

    lax.fori_loop(0, _RPT // 16, zb, 0)
    pltpu.sync_copy(zbuf, acc.at[pl.ds(s * _RPT, _RPT)])
    wid = c * _NS + s
    pltpu.sync_copy(row_hbm.at[wid], rowv)
    pltpu.sync_copy(w_hbm.at[wid], wv)
    plsc.subcore_barrier()

    def body(j, carry):
        pltpu.sync_copy(wv.at[j], acc.at[rowv.at[j]], add=True)
        return carry

    lax.fori_loop(0, _CPW, body, 0)
    plsc.subcore_barrier()
    pltpu.sync_copy(acc.at[pl.ds(s * _RPT, _RPT)],
                    out_hbm.at[c, pl.ds(s * _RPT, _RPT)])


_deg_call = pl.kernel(
    _deg_body,
    out_type=jax.ShapeDtypeStruct((_NC, _NP), jnp.float32),
    mesh=_mesh,
    scratch_types=[
        pltpu.VMEM((_CPW, _CH), jnp.int32),
        pltpu.VMEM((_CPW, _CH), jnp.float32),
        pltpu.VMEM((_RPT,), jnp.float32),
        pltpu.VMEM_SHARED((_NP,), jnp.float32),
    ],
)


_NBUF = 5


def _agg_body(hs_hbm, row_hbm, col_hbm, w_hbm, out_hbm,
              rowv, colv, wv, zbuf, acc, gbufs, sbufs, gsem, ssem):
    c = lax.axis_index("c")
    s = lax.axis_index("s")

    def zb(i, carry):
        zbuf[i, :] = jnp.zeros((_H,), jnp.float32)
        return carry

    lax.fori_loop(0, _RPT, zb, 0)
    pltpu.sync_copy(zbuf, acc.at[pl.ds(s * _RPT, _RPT)])
    wid = c * _NS + s
    pltpu.sync_copy(row_hbm.at[wid], rowv)
    pltpu.sync_copy(col_hbm.at[wid], colv)
    pltpu.sync_copy(w_hbm.at[wid], wv)
    plsc.subcore_barrier()

    def issue_gather(j, b):
        pltpu.async_copy(hs_hbm.at[rowv.at[j]], gbufs.at[b], gsem.at[b])

    def wait_gather(j, b):
        pltpu.make_async_copy(
            hs_hbm.at[rowv.at[j]], gbufs.at[b], gsem.at[b]).wait()

    def issue_scatter(j, b):
        pltpu.async_copy(sbufs.at[b], acc.at[colv.at[j]], ssem.at[b],
                         add=True)

    def wait_scatter(j, b):
        pltpu.make_async_copy(
            sbufs.at[b], acc.at[colv.at[j]], ssem.at[b]).wait()

    def scale(j, b):
        for grp in range(_CH // 16):
            wvec = wv[j, pl.ds(grp * 16, 16)]
            for k in range(16):
                e = grp * 16 + k
                sbufs[b, e, :] = gbufs[b, e, :] * wvec[k]

    for b in range(_NBUF):
        issue_gather(b, b)
    for b in range(_NBUF):
        wait_gather(b, b)
        scale(b, b)
        issue_scatter(b, b)
        issue_gather(b + _NBUF, b)

    def body(jj, carry):
        for b in range(_NBUF):
            j = jj * _NBUF + b
            wait_gather(j, b)
            wait_scatter(j - _NBUF, b)
            scale(j, b)
            issue_scatter(j, b)
            issue_gather(j + _NBUF, b)
        return carry

    lax.fori_loop(1, _CPW // _NBUF - 1, body, 0)

    for b in range(_NBUF):
        j = _CPW - _NBUF + b
        wait_gather(j, b)
        wait_scatter(j - _NBUF, b)
        scale(j, b)
        issue_scatter(j, b)
    for b in range(_NBUF):
        wait_scatter(_CPW - _NBUF + b, b)

    plsc.subcore_barrier()
    pltpu.sync_copy(acc.at[pl.ds(s * _RPT, _RPT)],
                    out_hbm.at[c, pl.ds(s * _RPT, _RPT)])


_agg_call = pl.kernel(
    _agg_body,
    out_type=jax.ShapeDtypeStruct((_NC, _NP, _H), jnp.float32),
    mesh=_mesh,
    compiler_params=pltpu.CompilerParams(use_tc_tiling_on_sc=False),
    scratch_types=[
        pltpu.VMEM((_CPW, _CH), jnp.int32),
        pltpu.VMEM((_CPW, _CH), jnp.int32),
        pltpu.VMEM((_CPW, _CH), jnp.float32),
        pltpu.VMEM((_RPT, _H), jnp.float32),
        pltpu.VMEM_SHARED((_NP, _H), jnp.float32),
        pltpu.VMEM((_NBUF, _CH, _H), jnp.float32),
        pltpu.VMEM((_NBUF, _CH, _H), jnp.float32),
        pltpu.SemaphoreType.DMA((_NBUF,)),
        pltpu.SemaphoreType.DMA((_NBUF,)),
    ],
)


def _tc1_body(x_ref, w0_ref, dp_ref, hs_ref, g16_ref):
    deg = dp_ref[0, :_N] + dp_ref[1, :_N] + 1.0
    g = lax.rsqrt(deg)[:, None]
    h = jnp.dot(x_ref[...], w0_ref[...], preferred_element_type=jnp.float32)
    hs_ref[...] = h * g
    g16_ref[...] = jnp.broadcast_to(g, (_N, _H))


_tc1_call = pl.pallas_call(
    _tc1_body,
    out_shape=(jax.ShapeDtypeStruct((_N, _H), jnp.float32),
               jax.ShapeDtypeStruct((_N, _H), jnp.float32)),
)


def _tc2_body(ap_ref, hs_ref, g16_ref, w1_ref, b0_ref, hs2_ref):
    aggs = ap_ref[0, :_N, :] + ap_ref[1, :_N, :]
    out1 = jnp.maximum(
        g16_ref[...] * (aggs + hs_ref[...]) + b0_ref[...], 0.0)
    h2 = jnp.dot(out1, w1_ref[...], preferred_element_type=jnp.float32)
    hs2_ref[...] = h2 * g16_ref[...]


_tc2_call = pl.pallas_call(
    _tc2_body,
    out_shape=jax.ShapeDtypeStruct((_N, _H), jnp.float32),
)


def _tc3_body(qp_ref, hs2_ref, g16_ref, b1_ref, out_ref):
    aggs = qp_ref[0, :_N, :] + qp_ref[1, :_N, :]
    out_ref[...] = g16_ref[...] * (aggs + hs2_ref[...]) + b1_ref[...]


_tc3_call = pl.pallas_call(
    _tc3_body,
    out_shape=jax.ShapeDtypeStruct((_N, _H), jnp.float32),
)


def kernel(x, edge_index, edge_weight, W0, b0, W1, b1):
    row = edge_index[0].reshape(_NW, _CPW, _CH)
    col = edge_index[1].reshape(_NW, _CPW, _CH)
    w = edge_weight.reshape(_NW, _CPW, _CH)
    w1p = jnp.zeros((_H, _H), jnp.float32).at[:, : W1.shape[1]].set(W1)
    b1p = jnp.zeros((_H,), jnp.float32).at[: b1.shape[0]].set(b1)

    dpart = _deg_call(row, w)
    hs, g16 = _tc1_call(x, W0, dpart)
    ap = _agg_call(hs, row, col, w)
    hs2 = _tc2_call(ap, hs, g16, w1p, b0)
    qp = _agg_call(hs2, row, col, w)
    outp = _tc3_call(qp, hs2, g16, b1p)
    return outp[:, : W1.shape[1]]


# speedup vs baseline: 59.1803x; 1.0898x over previous
import functools

import jax
import jax.numpy as jnp
from jax import lax
from jax.experimental import pallas as pl
from jax.experimental.pallas import tpu as pltpu
from jax.experimental.pallas import tpu_sc as plsc

_N = 10000
_E = 320000
_D = 128
_H = 16
_NC = 2
_NS = 16
_NW = _NC * _NS
_EPW = _E // _NW
_CH = 80
_CPW = _EPW // _CH
_NP = 10240
_RPT = _NP // _NS

_mesh = plsc.VectorSubcoreMesh(core_axis_name="c", subcore_axis_name="s")


def _deg_body(row_hbm, w_hbm, out_hbm, rowv, wv, zbuf, acc):
    c = lax.axis_index("c")
    s = lax.axis_index("s")

    def zb(i, carry):
        zbuf[pl.ds(i * 16, 16)] = jnp.zeros((16,), jnp.float32)
        return carry

    lax.fori_loop(0, _RPT // 16, zb, 0)
    pltpu.sync_copy(zbuf, acc.at[pl.ds(s * _RPT, _RPT)])
    wid = c * _NS + s
    pltpu.sync_copy(row_hbm.at[wid], rowv)
    pltpu.sync_copy(w_hbm.at[wid], wv)
    plsc.subcore_barrier()

    def body(j, carry):
        pltpu.sync_copy(wv.at[j], acc.at[rowv.at[j]], add=True)
        return carry

    lax.fori_loop(0, _CPW, body, 0)
    plsc.subcore_barrier()
    pltpu.sync_copy(acc.at[pl.ds(s * _RPT, _RPT)],
                    out_hbm.at[c, pl.ds(s * _RPT, _RPT)])


_deg_call = pl.kernel(
    _deg_body,
    out_type=jax.ShapeDtypeStruct((_NC, _NP), jnp.float32),
    mesh=_mesh,
    scratch_types=[
        pltpu.VMEM((_CPW, _CH), jnp.int32),
        pltpu.VMEM((_CPW, _CH), jnp.float32),
        pltpu.VMEM((_RPT,), jnp.float32),
        pltpu.VMEM_SHARED((_NP,), jnp.float32),
    ],
)


_NBUF = 5


def _agg_body(hs_hbm, row_hbm, col_hbm, w_hbm, out_hbm,
              rowv, colv, wv, zbuf, acc, hs_sh, gbufs, sbufs, gsem, ssem):
    c = lax.axis_index("c")
    s = lax.axis_index("s")

    def zb(i, carry):
        zbuf[i, :] = jnp.zeros((_H,), jnp.float32)
        return carry

    lax.fori_loop(0, _RPT, zb, 0)
    pltpu.sync_copy(zbuf, acc.at[pl.ds(s * _RPT, _RPT)])
    @pl.when(s < _NS - 1)
    def _():
        pltpu.sync_copy(hs_hbm.at[pl.ds(s * _RPT, _RPT)],
                        hs_sh.at[pl.ds(s * _RPT, _RPT)])
    @pl.when(s == _NS - 1)
    def _():
        pltpu.sync_copy(hs_hbm.at[pl.ds((_NS - 1) * _RPT, _N - (_NS - 1) * _RPT)],
                        hs_sh.at[pl.ds((_NS - 1) * _RPT, _N - (_NS - 1) * _RPT)])
    wid = c * _NS + s
    pltpu.sync_copy(row_hbm.at[wid], rowv)
    pltpu.sync_copy(col_hbm.at[wid], colv)
    pltpu.sync_copy(w_hbm.at[wid], wv)
    plsc.subcore_barrier()

    def issue_gather(j, b):
        pltpu.async_copy(hs_sh.at[rowv.at[j]], gbufs.at[b], gsem.at[b])

    def wait_gather(j, b):
        pltpu.make_async_copy(
            hs_sh.at[rowv.at[j]], gbufs.at[b], gsem.at[b]).wait()

    def issue_scatter(j, b):
        pltpu.async_copy(sbufs.at[b], acc.at[colv.at[j]], ssem.at[b],
                         add=True)

    def wait_scatter(j, b):
        pltpu.make_async_copy(
            sbufs.at[b], acc.at[colv.at[j]], ssem.at[b]).wait()

    def scale(j, b):
        for grp in range(_CH // 16):
            wvec = wv[j, pl.ds(grp * 16, 16)]
            for k in range(16):
                e = grp * 16 + k
                sbufs[b, e, :] = gbufs[b, e, :] * wvec[k]

    for b in range(_NBUF):
        issue_gather(b, b)
    for b in range(_NBUF):
        wait_gather(b, b)
        scale(b, b)
        issue_scatter(b, b)
        issue_gather(b + _NBUF, b)

    def body(jj, carry):
        for b in range(_NBUF):
            j = jj * _NBUF + b
            wait_gather(j, b)
            wait_scatter(j - _NBUF, b)
            scale(j, b)
            issue_scatter(j, b)
            issue_gather(j + _NBUF, b)
        return carry

    lax.fori_loop(1, _CPW // _NBUF - 1, body, 0)

    for b in range(_NBUF):
        j = _CPW - _NBUF + b
        wait_gather(j, b)
        wait_scatter(j - _NBUF, b)
        scale(j, b)
        issue_scatter(j, b)
    for b in range(_NBUF):
        wait_scatter(_CPW - _NBUF + b, b)

    plsc.subcore_barrier()
    pltpu.sync_copy(acc.at[pl.ds(s * _RPT, _RPT)],
                    out_hbm.at[c, pl.ds(s * _RPT, _RPT)])


_agg_call = pl.kernel(
    _agg_body,
    out_type=jax.ShapeDtypeStruct((_NC, _NP, _H), jnp.float32),
    mesh=_mesh,
    compiler_params=pltpu.CompilerParams(use_tc_tiling_on_sc=False),
    scratch_types=[
        pltpu.VMEM((_CPW, _CH), jnp.int32),
        pltpu.VMEM((_CPW, _CH), jnp.int32),
        pltpu.VMEM((_CPW, _CH), jnp.float32),
        pltpu.VMEM((_RPT, _H), jnp.float32),
        pltpu.VMEM_SHARED((_NP, _H), jnp.float32),
        pltpu.VMEM_SHARED((_NP, _H), jnp.float32),
        pltpu.VMEM((_NBUF, _CH, _H), jnp.float32),
        pltpu.VMEM((_NBUF, _CH, _H), jnp.float32),
        pltpu.SemaphoreType.DMA((_NBUF,)),
        pltpu.SemaphoreType.DMA((_NBUF,)),
    ],
)


def _tc1_body(x_ref, w0_ref, dp_ref, hs_ref, g16_ref):
    deg = dp_ref[0, :_N] + dp_ref[1, :_N] + 1.0
    g = lax.rsqrt(deg)[:, None]
    h = jnp.dot(x_ref[...], w0_ref[...], preferred_element_type=jnp.float32)
    hs_ref[...] = h * g
    g16_ref[...] = jnp.broadcast_to(g, (_N, _H))


_tc1_call = pl.pallas_call(
    _tc1_body,
    out_shape=(jax.ShapeDtypeStruct((_N, _H), jnp.float32),
               jax.ShapeDtypeStruct((_N, _H), jnp.float32)),
)


def _tc2_body(ap_ref, hs_ref, g16_ref, w1_ref, b0_ref, hs2_ref):
    aggs = ap_ref[0, :_N, :] + ap_ref[1, :_N, :]
    out1 = jnp.maximum(
        g16_ref[...] * (aggs + hs_ref[...]) + b0_ref[...], 0.0)
    h2 = jnp.dot(out1, w1_ref[...], preferred_element_type=jnp.float32)
    hs2_ref[...] = h2 * g16_ref[...]


_tc2_call = pl.pallas_call(
    _tc2_body,
    out_shape=jax.ShapeDtypeStruct((_N, _H), jnp.float32),
)


def _tc3_body(qp_ref, hs2_ref, g16_ref, b1_ref, out_ref):
    aggs = qp_ref[0, :_N, :] + qp_ref[1, :_N, :]
    out_ref[...] = g16_ref[...] * (aggs + hs2_ref[...]) + b1_ref[...]


_tc3_call = pl.pallas_call(
    _tc3_body,
    out_shape=jax.ShapeDtypeStruct((_N, _H), jnp.float32),
)


def kernel(x, edge_index, edge_weight, W0, b0, W1, b1):
    row = edge_index[0].reshape(_NW, _CPW, _CH)
    col = edge_index[1].reshape(_NW, _CPW, _CH)
    w = edge_weight.reshape(_NW, _CPW, _CH)
    w1p = jnp.zeros((_H, _H), jnp.float32).at[:, : W1.shape[1]].set(W1)
    b1p = jnp.zeros((_H,), jnp.float32).at[: b1.shape[0]].set(b1)

    dpart = _deg_call(row, w)
    hs, g16 = _tc1_call(x, W0, dpart)
    ap = _agg_call(hs, row, col, w)
    hs2 = _tc2_call(ap, hs, g16, w1p, b0)
    qp = _agg_call(hs2, row, col, w)
    outp = _tc3_call(qp, hs2, g16, b1p)
    return outp[:, : W1.shape[1]]


# speedup vs baseline: 65.8212x; 1.1122x over previous
import functools

import jax
import jax.numpy as jnp
from jax import lax
from jax.experimental import pallas as pl
from jax.experimental.pallas import tpu as pltpu
from jax.experimental.pallas import tpu_sc as plsc

_N = 10000
_E = 320000
_D = 128
_H = 16
_H2 = 7
_NC = 2
_NS = 16
_NW = _NC * _NS
_EPW = _E // _NW
_CH = 80
_CPW = _EPW // _CH
_NP = 10240
_RPT = _NP // _NS

_mesh = plsc.VectorSubcoreMesh(core_axis_name="c", subcore_axis_name="s")


def _deg_body(er_hbm, w_hbm, out_hbm, rowv, wv, zbuf, acc, dsem):
    c = lax.axis_index("c")
    s = lax.axis_index("s")

    def zb(i, carry):
        zbuf[pl.ds(i * 16, 16)] = jnp.zeros((16,), jnp.float32)
        return carry

    lax.fori_loop(0, _RPT // 16, zb, 0)
    pltpu.sync_copy(zbuf, acc.at[pl.ds(s * _RPT, _RPT)])
    wid = c * _NS + s
    pltpu.sync_copy(er_hbm.at[0, wid], rowv)
    pltpu.sync_copy(w_hbm.at[wid], wv)
    plsc.subcore_barrier()

    def body(j, carry):
        pltpu.async_copy(wv.at[j], acc.at[rowv.at[j]], dsem, add=True)
        return carry

    lax.fori_loop(0, _CPW, body, 0)

    def drain(j, carry):
        pltpu.make_async_copy(wv.at[j], acc.at[rowv.at[j]], dsem).wait()
        return carry

    lax.fori_loop(0, _CPW, drain, 0)
    plsc.subcore_barrier()
    pltpu.sync_copy(acc.at[pl.ds(s * _RPT, _RPT)],
                    out_hbm.at[c, pl.ds(s * _RPT, _RPT)])


_deg_call = pl.kernel(
    _deg_body,
    out_type=jax.ShapeDtypeStruct((_NC, _NP), jnp.float32),
    mesh=_mesh,
    scratch_types=[
        pltpu.VMEM((_CPW, _CH), jnp.int32),
        pltpu.VMEM((_CPW, _CH), jnp.float32),
        pltpu.VMEM((_RPT,), jnp.float32),
        pltpu.VMEM_SHARED((_NP,), jnp.float32),
        pltpu.SemaphoreType.DMA,
    ],
)


_NBUF = 5


def _agg_body(hs_hbm, er_hbm, w_hbm, out_hbm,
              rowv, colv, wv, zbuf, acc, hs_sh, gbufs, sbufs, gsem, ssem):
    c = lax.axis_index("c")
    s = lax.axis_index("s")

    def zb(i, carry):
        zbuf[i, :] = jnp.zeros((_H,), jnp.float32)
        return carry

    lax.fori_loop(0, _RPT, zb, 0)
    pltpu.sync_copy(zbuf, acc.at[pl.ds(s * _RPT, _RPT)])
    @pl.when(s < _NS - 1)
    def _():
        pltpu.sync_copy(hs_hbm.at[pl.ds(s * _RPT, _RPT)],
                        hs_sh.at[pl.ds(s * _RPT, _RPT)])
    @pl.when(s == _NS - 1)
    def _():
        pltpu.sync_copy(hs_hbm.at[pl.ds((_NS - 1) * _RPT, _N - (_NS - 1) * _RPT)],
                        hs_sh.at[pl.ds((_NS - 1) * _RPT, _N - (_NS - 1) * _RPT)])
    wid = c * _NS + s
    pltpu.sync_copy(er_hbm.at[0, wid], rowv)
    pltpu.sync_copy(er_hbm.at[1, wid], colv)
    pltpu.sync_copy(w_hbm.at[wid], wv)
    plsc.subcore_barrier()

    def issue_gather(j, b):
        pltpu.async_copy(hs_sh.at[rowv.at[j]], gbufs.at[b], gsem.at[b])

    def wait_gather(j, b):
        pltpu.make_async_copy(
            hs_sh.at[rowv.at[j]], gbufs.at[b], gsem.at[b]).wait()

    def issue_scatter(j, b):
        pltpu.async_copy(sbufs.at[b], acc.at[colv.at[j]], ssem.at[b],
                         add=True)

    def wait_scatter(j, b):
        pltpu.make_async_copy(
            sbufs.at[b], acc.at[colv.at[j]], ssem.at[b]).wait()

    def scale(j, b):
        for grp in range(_CH // 16):
            wvec = wv[j, pl.ds(grp * 16, 16)]
            for k in range(16):
                e = grp * 16 + k
                sbufs[b, e, :] = gbufs[b, e, :] * wvec[k]

    for b in range(_NBUF):
        issue_gather(b, b)
    for b in range(_NBUF):
        wait_gather(b, b)
        scale(b, b)
        issue_scatter(b, b)
        issue_gather(b + _NBUF, b)

    def body(jj, carry):
        for b in range(_NBUF):
            j = jj * _NBUF + b
            wait_gather(j, b)
            wait_scatter(j - _NBUF, b)
            scale(j, b)
            issue_scatter(j, b)
            issue_gather(j + _NBUF, b)
        return carry

    lax.fori_loop(1, _CPW // _NBUF - 1, body, 0)

    for b in range(_NBUF):
        j = _CPW - _NBUF + b
        wait_gather(j, b)
        wait_scatter(j - _NBUF, b)
        scale(j, b)
        issue_scatter(j, b)
    for b in range(_NBUF):
        wait_scatter(_CPW - _NBUF + b, b)

    plsc.subcore_barrier()
    pltpu.sync_copy(acc.at[pl.ds(s * _RPT, _RPT)],
                    out_hbm.at[c, pl.ds(s * _RPT, _RPT)])


_agg_call = pl.kernel(
    _agg_body,
    out_type=jax.ShapeDtypeStruct((_NC, _NP, _H), jnp.float32),
    mesh=_mesh,
    compiler_params=pltpu.CompilerParams(use_tc_tiling_on_sc=False),
    scratch_types=[
        pltpu.VMEM((_CPW, _CH), jnp.int32),
        pltpu.VMEM((_CPW, _CH), jnp.int32),
        pltpu.VMEM((_CPW, _CH), jnp.float32),
        pltpu.VMEM((_RPT, _H), jnp.float32),
        pltpu.VMEM_SHARED((_NP, _H), jnp.float32),
        pltpu.VMEM_SHARED((_NP, _H), jnp.float32),
        pltpu.VMEM((_NBUF, _CH, _H), jnp.float32),
        pltpu.VMEM((_NBUF, _CH, _H), jnp.float32),
        pltpu.SemaphoreType.DMA((_NBUF,)),
        pltpu.SemaphoreType.DMA((_NBUF,)),
    ],
)


def _tc1_body(x_ref, w0_ref, dp_ref, hs_ref, g16_ref):
    deg = dp_ref[0, :_N] + dp_ref[1, :_N] + 1.0
    g = lax.rsqrt(deg)[:, None]
    h = jnp.dot(x_ref[...], w0_ref[...], preferred_element_type=jnp.float32)
    hs_ref[...] = h * g
    g16_ref[...] = jnp.broadcast_to(g, (_N, _H))


_tc1_call = pl.pallas_call(
    _tc1_body,
    out_shape=(jax.ShapeDtypeStruct((_N, _H), jnp.float32),
               jax.ShapeDtypeStruct((_N, _H), jnp.float32)),
)


def _tc2_body(ap_ref, hs_ref, g16_ref, w1_ref, b0_ref, hs2_ref):
    aggs = ap_ref[0, :_N, :] + ap_ref[1, :_N, :]
    out1 = jnp.maximum(
        g16_ref[...] * (aggs + hs_ref[...]) + b0_ref[...], 0.0)
    w1p = jnp.concatenate(
        [w1_ref[...], jnp.zeros((_H, _H - _H2), jnp.float32)], axis=1)
    h2 = jnp.dot(out1, w1p, preferred_element_type=jnp.float32)
    hs2_ref[...] = h2 * g16_ref[...]


_tc2_call = pl.pallas_call(
    _tc2_body,
    out_shape=jax.ShapeDtypeStruct((_N, _H), jnp.float32),
)


def _tc3_body(qp_ref, hs2_ref, g16_ref, b1_ref, out_ref):
    aggs = qp_ref[0, :_N, :_H2] + qp_ref[1, :_N, :_H2]
    out_ref[...] = (g16_ref[:, :_H2] * (aggs + hs2_ref[:, :_H2])
                    + b1_ref[...])


_tc3_call = pl.pallas_call(
    _tc3_body,
    out_shape=jax.ShapeDtypeStruct((_N, _H2), jnp.float32),
)


def kernel(x, edge_index, edge_weight, W0, b0, W1, b1):
    er = edge_index.reshape(2, _NW, _CPW, _CH)
    w = edge_weight.reshape(_NW, _CPW, _CH)

    dpart = _deg_call(er, w)
    hs, g16 = _tc1_call(x, W0, dpart)
    ap = _agg_call(hs, er, w)
    hs2 = _tc2_call(ap, hs, g16, W1, b0)
    qp = _agg_call(hs2, er, w)
    return _tc3_call(qp, hs2, g16, b1)
